# Initial kernel scaffold; baseline (speedup 1.0000x reference)
import jax
import jax.numpy as jnp
from jax.experimental import pallas as pl


def kernel(x, W, b):
    raise NotImplementedError("write your pallas kernel here")


# speedup vs baseline: 4.4114x; 4.4114x over previous
import jax
import jax.numpy as jnp
from jax import lax
from jax.experimental import pallas as pl

_STRIDE = 32.0
_AW = (116.0, 156.0, 373.0)
_AH = (90.0, 198.0, 326.0)


def _decode(z, f, n_ch, n_anchors):
    col = lax.broadcasted_iota(jnp.int32, z.shape, 1)
    row = lax.broadcasted_iota(jnp.int32, z.shape, 0)
    ch = col % n_ch
    xs = (row % f).astype(jnp.float32)
    ys = (row // f).astype(jnp.float32)
    sig = jax.nn.sigmoid(z)
    e = jnp.exp(z)
    wa = jnp.where(col < n_ch, _AW[0], jnp.where(col < 2 * n_ch, _AW[1], _AW[2]))
    ha = jnp.where(col < n_ch, _AH[0], jnp.where(col < 2 * n_ch, _AH[1], _AH[2]))
    return jnp.where(
        ch == 0, (sig + xs) * _STRIDE,
        jnp.where(
            ch == 1, (sig + ys) * _STRIDE,
            jnp.where(ch == 2, e * wa, jnp.where(ch == 3, e * ha, sig))))


def _body(x_ref, wt_ref, b_ref, o_ref, *, f, n_ch, n_anchors):
    xb = x_ref[0]
    wt = wt_ref[...]
    z = lax.dot_general(xb, wt, (((0,), (0,)), ((), ())),
                        preferred_element_type=jnp.float32)
    z = z + b_ref[...]
    out = _decode(z, f, n_ch, n_anchors)
    for a in range(n_anchors):
        o_ref[0, a] = out[:, a * n_ch:(a + 1) * n_ch]


def kernel(x, W, b):
    B, C, f, _ = x.shape
    n_anchors, n_ch = 3, 85
    hw = f * f
    oc = n_anchors * n_ch
    xr = x.reshape(B, C, hw)
    wt = W.T
    b2 = b.reshape(1, oc)

    import functools
    body = functools.partial(_body, f=f, n_ch=n_ch, n_anchors=n_anchors)
    out = pl.pallas_call(
        body,
        grid=(B,),
        in_specs=[
            pl.BlockSpec((1, C, hw), lambda i: (i, 0, 0)),
            pl.BlockSpec((C, oc), lambda i: (0, 0)),
            pl.BlockSpec((1, oc), lambda i: (0, 0)),
        ],
        out_specs=pl.BlockSpec((1, n_anchors, hw, n_ch),
                               lambda i: (i, 0, 0, 0)),
        out_shape=jax.ShapeDtypeStruct((B, n_anchors, hw, n_ch), jnp.float32),
    )(xr, wt, b2)
    return out.reshape(B, n_anchors * hw, n_ch)


# speedup vs baseline: 4.4406x; 1.0066x over previous
import jax
import jax.numpy as jnp
from jax import lax
from jax.experimental import pallas as pl

_STRIDE = 32.0
_AW = (116.0, 156.0, 373.0)
_AH = (90.0, 198.0, 326.0)


def _decode(z, f, n_ch, n_anchors):
    col = lax.broadcasted_iota(jnp.int32, z.shape, 1)
    row = lax.broadcasted_iota(jnp.int32, z.shape, 0)
    ch = col % n_ch
    xs = (row % f).astype(jnp.float32)
    ys = (row // f).astype(jnp.float32)
    sig = jax.nn.sigmoid(z)
    e = jnp.exp(z)
    wa = jnp.where(col < n_ch, _AW[0], jnp.where(col < 2 * n_ch, _AW[1], _AW[2]))
    ha = jnp.where(col < n_ch, _AH[0], jnp.where(col < 2 * n_ch, _AH[1], _AH[2]))
    return jnp.where(
        ch == 0, (sig + xs) * _STRIDE,
        jnp.where(
            ch == 1, (sig + ys) * _STRIDE,
            jnp.where(ch == 2, e * wa, jnp.where(ch == 3, e * ha, sig))))


def _body(x_ref, wt_ref, b_ref, o_ref, *, f, n_ch, n_anchors):
    xb = x_ref[0].astype(jnp.bfloat16)
    wt = wt_ref[...].astype(jnp.bfloat16)
    z = lax.dot_general(xb, wt, (((0,), (0,)), ((), ())),
                        preferred_element_type=jnp.float32)
    z = z + b_ref[...]
    out = _decode(z, f, n_ch, n_anchors)
    for a in range(n_anchors):
        o_ref[0, a] = out[:, a * n_ch:(a + 1) * n_ch]


def kernel(x, W, b):
    B, C, f, _ = x.shape
    n_anchors, n_ch = 3, 85
    hw = f * f
    oc = n_anchors * n_ch
    xr = x.reshape(B, C, hw)
    wt = W.T
    b2 = b.reshape(1, oc)

    import functools
    body = functools.partial(_body, f=f, n_ch=n_ch, n_anchors=n_anchors)
    out = pl.pallas_call(
        body,
        grid=(B,),
        in_specs=[
            pl.BlockSpec((1, C, hw), lambda i: (i, 0, 0)),
            pl.BlockSpec((C, oc), lambda i: (0, 0)),
            pl.BlockSpec((1, oc), lambda i: (0, 0)),
        ],
        out_specs=pl.BlockSpec((1, n_anchors, hw, n_ch),
                               lambda i: (i, 0, 0, 0)),
        out_shape=jax.ShapeDtypeStruct((B, n_anchors, hw, n_ch), jnp.float32),
    )(xr, wt, b2)
    return out.reshape(B, n_anchors * hw, n_ch)
